# baseline (speedup 1.0000x reference)
import dataclasses
import functools

import jax
import jax.numpy as jnp
from jax import lax
from jax.experimental import pallas as pl
from jax.experimental.pallas import tpu as pltpu
from jax.experimental.pallas import tpu_sc as plsc

_NUM_SC_CORES = 2
_NUM_SC_SUBCORES = 16
_LANES = 16


def _sc_gather(gmf_emb, mlp_emb, i0, i1, wg):
    B = i0.shape[0]
    D = gmf_emb.shape[1]
    nw = _NUM_SC_CORES * _NUM_SC_SUBCORES
    b_per_w = B // nw
    assert B % (8 * nw) == 0
    chunk = 64
    n_par = 3
    n_chunks = b_per_w // chunk
    assert n_chunks >= 2 and b_per_w % chunk == 0
    mesh = plsc.VectorSubcoreMesh(core_axis_name="c", subcore_axis_name="s")
    row_t = jax.ShapeDtypeStruct((B, D), jnp.float32)
    gp_t = jax.ShapeDtypeStruct((_LANES, B), jnp.float32)

    buf_t = pltpu.VMEM((chunk, D), jnp.float32)
    gpbuf_t = pltpu.VMEM((_LANES, b_per_w), jnp.float32)
    n_bufs = 4 * n_par

    sc_params = pltpu.CompilerParams()
    if "needs_layout_passes" in pltpu.CompilerParams.__dataclass_fields__:
        sc_params = dataclasses.replace(sc_params, needs_layout_passes=False)

    @functools.partial(
        pl.kernel,
        mesh=mesh,
        compiler_params=sc_params,
        out_type=[gp_t, row_t, row_t],
        scratch_types=[
            pltpu.VMEM((b_per_w,), jnp.int32),
            pltpu.VMEM((b_per_w,), jnp.int32),
            pltpu.VMEM((D,), jnp.float32),
            gpbuf_t,
        ] + [buf_t] * n_bufs
          + [pltpu.SemaphoreType.DMA] * (n_bufs + 2 * n_par + 1),
    )
    def gather_kernel(gmf_hbm, mlp_hbm, i0_hbm, i1_hbm, wg_hbm,
                      gp_hbm, m0_hbm, m1_hbm,
                      idx0_v, idx1_v, wg_v, gpb, *rest):
        bufs = rest[:n_bufs]
        g_sems = rest[n_bufs:2 * n_bufs]
        w_sems = rest[2 * n_bufs:]
        names = ("g0", "g1", "m0", "m1")

        def bi(s, q):
            return names.index(s) * n_par + q

        wnames = ("m0", "m1")

        def wsem(s, q):
            return w_sems[wnames.index(s) * n_par + q]

        wid = lax.axis_index("s") * _NUM_SC_CORES + lax.axis_index("c")
        base = wid * b_per_w
        pltpu.sync_copy(i0_hbm.at[pl.ds(base, b_per_w)], idx0_v)
        pltpu.sync_copy(i1_hbm.at[pl.ds(base, b_per_w)], idx1_v)
        pltpu.sync_copy(wg_hbm, wg_v)
        wgv = [wg_v.at[pl.ds(l, _LANES)][...] for l in range(0, D, _LANES)]
        lane_ids = lax.iota(jnp.int32, _LANES)

        gh = {}
        wh = {}

        def start_gathers(c):
            q = c % n_par
            off = c * chunk
            s0 = idx0_v.at[pl.ds(off, chunk)]
            s1 = idx1_v.at[pl.ds(off, chunk)]
            for s, tbl, idx in (("g0", gmf_hbm, s0), ("g1", gmf_hbm, s1),
                                ("m0", mlp_hbm, s0), ("m1", mlp_hbm, s1)):
                b = bi(s, q)
                gh[(s, c)] = pltpu.async_copy(tbl.at[idx], bufs[b],
                                              g_sems[b])

        for c in range(min(n_par, n_chunks)):
            start_gathers(c)
        for c in range(n_chunks):
            q = c % n_par
            off = c * chunk
            osl = pl.ds(base + off, chunk)
            gh[("g0", c)].wait()
            gh[("g1", c)].wait()
            b0 = bufs[bi("g0", q)]
            b1 = bufs[bi("g1", q)]

            @pl.loop(0, chunk)
            def _(r):
                acc = None
                for li, l in enumerate(range(0, D, _LANES)):
                    sl2 = pl.ds(l, _LANES)
                    term = b0.at[r, sl2][...] * b1.at[r, sl2][...] * wgv[li]
                    acc = term if acc is None else acc + term
                col = jnp.full((_LANES,), off + r, jnp.int32)
                plsc.store_scatter(gpb, [lane_ids, col], acc)

            gh[("m0", c)].wait()
            wh[("m0", c)] = pltpu.async_copy(bufs[bi("m0", q)],
                                             m0_hbm.at[osl], wsem("m0", q))
            gh[("m1", c)].wait()
            wh[("m1", c)] = pltpu.async_copy(bufs[bi("m1", q)],
                                             m1_hbm.at[osl], wsem("m1", q))
            if c + n_par < n_chunks:
                for s in wnames:
                    wh[(s, c)].wait()
                start_gathers(c + n_par)
        gp_wh = pltpu.async_copy(gpb, gp_hbm.at[:, pl.ds(base, b_per_w)],
                                 w_sems[2 * n_par])
        for c in range(max(0, n_chunks - n_par), n_chunks):
            for s in wnames:
                wh[(s, c)].wait()
        gp_wh.wait()

    return gather_kernel(gmf_emb, mlp_emb, i0, i1, wg)


def _tc_body(gp_r, m0_r, m1_r, w1a_r, w1b_r, b1_r, w2_r, b2_r,
             w3_r, b3_r, wm_r, bout_r, o_r):
    bf = jnp.bfloat16

    def mm(a, b):
        return jnp.dot(a.astype(bf), b.astype(bf),
                       preferred_element_type=jnp.float32)

    h = mm(m0_r[...], w1a_r[...]) + mm(m1_r[...], w1b_r[...])
    h = jnp.maximum(h + b1_r[...], 0.0)
    h = jnp.maximum(mm(h, w2_r[...]) + b2_r[...], 0.0)
    h = jnp.maximum(mm(h, w3_r[...]) + b3_r[...], 0.0)
    dn = (((1,), (1,)), ((), ()))
    s = (jnp.sum(gp_r[...], axis=0, keepdims=True)
         + lax.dot_general(wm_r[...], h, dn,
                           preferred_element_type=jnp.float32)
         + bout_r[...])
    o_r[...] = jax.nn.sigmoid(s)


def _tc_mlp(gp, m0, m1, W1, b1, W2, b2, W3, b3, Wout, bout):
    B, D = m0.shape
    blk = 2048
    w1a = W1[:D]
    w1b = W1[D:]
    wm = Wout[D:].reshape(1, -1)
    grid = (B // blk,)

    def batch_spec():
        return pl.BlockSpec((blk, D), lambda i: (i, 0))

    def full_spec(shape):
        return pl.BlockSpec(shape, lambda i: tuple(0 for _ in shape))

    return pl.pallas_call(
        _tc_body,
        grid=grid,
        in_specs=[
            pl.BlockSpec((_LANES, blk), lambda i: (0, i)),
            batch_spec(), batch_spec(),
            full_spec(w1a.shape), full_spec(w1b.shape),
            full_spec((1, b1.shape[0])),
            full_spec(W2.shape), full_spec((1, b2.shape[0])),
            full_spec(W3.shape), full_spec((1, b3.shape[0])),
            full_spec(wm.shape),
            full_spec((1, 1)),
        ],
        out_specs=pl.BlockSpec((1, blk), lambda i: (0, i)),
        out_shape=jax.ShapeDtypeStruct((1, B), jnp.float32),
        compiler_params=pltpu.CompilerParams(
            dimension_semantics=("parallel",),
        ),
    )(gp, m0, m1, w1a, w1b, b1.reshape(1, -1), W2, b2.reshape(1, -1),
      W3, b3.reshape(1, -1), wm, bout.reshape(1, 1)).reshape(B, 1)


def kernel(x, gmf_emb, mlp_emb, W1, b1, W2, b2, W3, b3, Wout, bout):
    B = x.shape[0]
    D = gmf_emb.shape[1]
    i0 = x[:, 0]
    i1 = x[:, 1]
    wg = Wout[:D, 0]
    n_rounds = 2
    h = B // n_rounds
    outs = []
    for r in range(n_rounds):
        sl = slice(r * h, (r + 1) * h)
        gp, m0, m1 = _sc_gather(gmf_emb, mlp_emb, i0[sl], i1[sl], wg)
        outs.append(_tc_mlp(gp, m0, m1, W1, b1, W2, b2, W3, b3, Wout, bout))
    return jnp.concatenate(outs, axis=0)
